# baseline (speedup 1.0000x reference)
import jax
import jax.numpy as jnp
from jax.experimental import pallas as pl
from jax.experimental.pallas import tpu as pltpu

_TM = 512
_TN = 2048
_CHUNKS = 8
_CROWS = _TN // _CHUNKS


def _linear_kernel(x_ref, w_hbm, b_ref, o_ref, w_vmem, sems):
    j = pl.program_id(0)
    i = pl.program_id(1)

    def chunk_copy(c):
        return pltpu.make_async_copy(
            w_hbm.at[pl.ds(j * _TN + c * _CROWS, _CROWS), :],
            w_vmem.at[pl.ds(c * _CROWS, _CROWS), :],
            sems.at[c],
        )

    @pl.when(i == 0)
    def _():
        for c in range(_CHUNKS):
            chunk_copy(c).start()

    x = x_ref[...]

    @pl.when(i == 0)
    def _():
        for c in range(_CHUNKS):
            chunk_copy(c).wait()
            sl = pl.ds(c * _CROWS, _CROWS)
            acc = jax.lax.dot_general(
                x, w_vmem[sl, :],
                dimension_numbers=(((1,), (1,)), ((), ())),
                preferred_element_type=jnp.float32,
            )
            o_ref[:, sl] = acc + b_ref[:, sl]

    @pl.when(i > 0)
    def _():
        acc = jax.lax.dot_general(
            x, w_vmem[...],
            dimension_numbers=(((1,), (1,)), ((), ())),
            preferred_element_type=jnp.float32,
        )
        o_ref[...] = acc + b_ref[...]


def kernel(x, weight, bias):
    B, K = x.shape
    N, _ = weight.shape
    gm = B // _TM
    gn = N // _TN

    b2 = bias.reshape(1, N)

    return pl.pallas_call(
        _linear_kernel,
        grid=(gn, gm),
        in_specs=[
            pl.BlockSpec((_TM, K), lambda j, i: (i, 0)),
            pl.BlockSpec(memory_space=pl.ANY),
            pl.BlockSpec((1, _TN), lambda j, i: (0, j)),
        ],
        out_specs=pl.BlockSpec((_TM, _TN), lambda j, i: (i, j)),
        out_shape=jax.ShapeDtypeStruct((B, N), x.dtype),
        scratch_shapes=[
            pltpu.VMEM((_TN, K), jnp.float32),
            pltpu.SemaphoreType.DMA((_CHUNKS,)),
        ],
        compiler_params=pltpu.CompilerParams(
            dimension_semantics=("arbitrary", "arbitrary"),
            vmem_limit_bytes=64 * 1024 * 1024,
        ),
        cost_estimate=pl.CostEstimate(
            flops=2 * B * K * N, transcendentals=0,
            bytes_accessed=B * K * 4 + N * K * 4 + B * N * 4,
        ),
    )(x, weight, b2)


# speedup vs baseline: 1.0606x; 1.0606x over previous
import jax
import jax.numpy as jnp
from jax.experimental import pallas as pl
from jax.experimental.pallas import tpu as pltpu


def _linear_kernel(x_ref, w_ref, b_ref, o_ref):
    acc = jax.lax.dot_general(
        x_ref[...], w_ref[...],
        dimension_numbers=(((1,), (1,)), ((), ())),
        preferred_element_type=jnp.float32,
    )
    o_ref[...] = acc + b_ref[...]


def kernel(x, weight, bias):
    B, K = x.shape
    N, _ = weight.shape
    tm = 512
    tn = 1024
    gm = B // tm
    gn = N // tn

    b2 = bias.reshape(1, N)

    return pl.pallas_call(
        _linear_kernel,
        grid=(gn, gm),
        in_specs=[
            pl.BlockSpec((tm, K), lambda j, i: (i, 0)),
            pl.BlockSpec((tn, K), lambda j, i: (j, 0)),
            pl.BlockSpec((1, tn), lambda j, i: (0, j)),
        ],
        out_specs=pl.BlockSpec((tm, tn), lambda j, i: (i, j)),
        out_shape=jax.ShapeDtypeStruct((B, N), x.dtype),
        compiler_params=pltpu.CompilerParams(
            dimension_semantics=("arbitrary", "arbitrary"),
            vmem_limit_bytes=60000 * 1024,
        ),
        cost_estimate=pl.CostEstimate(
            flops=2 * B * K * N, transcendentals=0,
            bytes_accessed=B * K * 4 + N * K * 4 + B * N * 4,
        ),
    )(x, weight, b2)
